# baseline (speedup 1.0000x reference)
import math

import jax
import jax.numpy as jnp
import numpy as np
from jax.experimental import pallas as pl

D = 64
B = 16384
_HALF = D // 2

def _fused_kernel(x_ref, c_ref, t_ref, Wstep_ref, bstep_ref,
                  W1_ref, b1_ref, W2_ref, b2_ref, out_ref):
    t_col = t_ref[:]
    bx = t_col.shape[0]
    col = jax.lax.broadcasted_iota(jnp.int32, (bx, D), 1)
    fidx = jnp.where(col < _HALF, col, col - _HALF).astype(jnp.float32)
    freqs = jnp.exp(fidx * (-math.log(10000.0) / _HALF))
    args = t_col * freqs
    temb = jnp.where(col < _HALF, jnp.cos(args), jnp.sin(args))

    temb = jnp.dot(temb, Wstep_ref[:], preferred_element_type=jnp.float32)
    temb = temb + bstep_ref[:]

    h = jnp.dot(x_ref[:], W1_ref[0:D, :], preferred_element_type=jnp.float32)
    h += jnp.dot(c_ref[:], W1_ref[D:2 * D, :], preferred_element_type=jnp.float32)
    h += jnp.dot(temb, W1_ref[2 * D:3 * D, :], preferred_element_type=jnp.float32)
    h += b1_ref[:]
    h = jnp.where(h > 0, h, 0.01 * h)

    out = jnp.dot(h, W2_ref[:], preferred_element_type=jnp.float32)
    out_ref[:] = out + b2_ref[:]


@jax.jit
def kernel(x, t, c, W_step, b_step, W1, b1, W2, b2):
    bx = 2048
    grid = (B // bx,)

    tf = t.astype(jnp.float32).reshape(B, 1)
    out = pl.pallas_call(
        _fused_kernel,
        grid=grid,
        in_specs=[
            pl.BlockSpec((bx, D), lambda i: (i, 0)),
            pl.BlockSpec((bx, D), lambda i: (i, 0)),
            pl.BlockSpec((bx, 1), lambda i: (i, 0)),
            pl.BlockSpec((D, D), lambda i: (0, 0)),
            pl.BlockSpec((1, D), lambda i: (0, 0)),
            pl.BlockSpec((3 * D, 3 * D), lambda i: (0, 0)),
            pl.BlockSpec((1, 3 * D), lambda i: (0, 0)),
            pl.BlockSpec((3 * D, D), lambda i: (0, 0)),
            pl.BlockSpec((1, D), lambda i: (0, 0)),
        ],
        out_specs=pl.BlockSpec((bx, D), lambda i: (i, 0)),
        out_shape=jax.ShapeDtypeStruct((B, D), jnp.float32),
    )(x, c, tf, W_step, b_step.reshape(1, D),
      W1, b1.reshape(1, 3 * D), W2, b2.reshape(1, D))
    return out


# speedup vs baseline: 1.5274x; 1.5274x over previous
import math

import jax
import jax.numpy as jnp
import numpy as np
from jax.experimental import pallas as pl

D = 64
B = 16384
_HALF = D // 2

def _fused_kernel(x_ref, c_ref, t_ref, Wstep_ref, bstep_ref,
                  W1_ref, b1_ref, W2_ref, b2_ref, out_ref):
    t_col = t_ref[:]
    bx = t_col.shape[0]
    col = jax.lax.broadcasted_iota(jnp.int32, (bx, D), 1)
    fidx = jnp.where(col < _HALF, col, col - _HALF).astype(jnp.float32)
    freqs = jnp.exp(fidx * (-math.log(10000.0) / _HALF))
    shift = jnp.where(col < _HALF, 0.0, math.pi / 2).astype(jnp.float32)
    args = t_col * freqs - shift
    n = jnp.round(args * jnp.float32(1.0 / (2.0 * math.pi)))
    r = args - n * jnp.float32(6.28125)
    r = r - n * jnp.float32(1.9353071795864769e-03)
    s = r * r
    temb = jnp.float32(-2.2398469402767916e-07)
    temb = temb * s + jnp.float32(2.430807671139143e-05)
    temb = temb * s + jnp.float32(-1.3867885560937686e-03)
    temb = temb * s + jnp.float32(4.1662991555676473e-02)
    temb = temb * s + jnp.float32(-4.999981914909368e-01)
    temb = temb * s + jnp.float32(1.0)

    temb = jnp.dot(temb, Wstep_ref[:], preferred_element_type=jnp.float32)
    temb = temb + bstep_ref[:]

    h = jnp.dot(x_ref[:], W1_ref[0:D, :], preferred_element_type=jnp.float32)
    h += jnp.dot(c_ref[:], W1_ref[D:2 * D, :], preferred_element_type=jnp.float32)
    h += jnp.dot(temb, W1_ref[2 * D:3 * D, :], preferred_element_type=jnp.float32)
    h += b1_ref[:]
    h = jnp.where(h > 0, h, 0.01 * h)

    out = jnp.dot(h, W2_ref[:], preferred_element_type=jnp.float32)
    out_ref[:] = out + b2_ref[:]


@jax.jit
def kernel(x, t, c, W_step, b_step, W1, b1, W2, b2):
    bx = 2048
    grid = (B // bx,)

    tf = t.astype(jnp.float32).reshape(B, 1)
    out = pl.pallas_call(
        _fused_kernel,
        grid=grid,
        in_specs=[
            pl.BlockSpec((bx, D), lambda i: (i, 0)),
            pl.BlockSpec((bx, D), lambda i: (i, 0)),
            pl.BlockSpec((bx, 1), lambda i: (i, 0)),
            pl.BlockSpec((D, D), lambda i: (0, 0)),
            pl.BlockSpec((1, D), lambda i: (0, 0)),
            pl.BlockSpec((3 * D, 3 * D), lambda i: (0, 0)),
            pl.BlockSpec((1, 3 * D), lambda i: (0, 0)),
            pl.BlockSpec((3 * D, D), lambda i: (0, 0)),
            pl.BlockSpec((1, D), lambda i: (0, 0)),
        ],
        out_specs=pl.BlockSpec((bx, D), lambda i: (i, 0)),
        out_shape=jax.ShapeDtypeStruct((B, D), jnp.float32),
    )(x, c, tf, W_step, b_step.reshape(1, D),
      W1, b1.reshape(1, 3 * D), W2, b2.reshape(1, D))
    return out


# speedup vs baseline: 1.8266x; 1.1959x over previous
import math

import jax
import jax.numpy as jnp
import numpy as np
from jax.experimental import pallas as pl

D = 64
B = 16384
_HALF = D // 2

def _fused_kernel(x_ref, c_ref, t_ref, Wstep_ref, bstep_ref,
                  W1_ref, b1_ref, W2_ref, b2_ref, out_ref):
    t_row = t_ref[:]
    row = jax.lax.broadcasted_iota(jnp.int32, (D, 1), 0)
    fidx = jnp.where(row < _HALF, row, row - _HALF).astype(jnp.float32)
    freq_col = jnp.exp(fidx * (-math.log(10000.0) / _HALF))
    shift_col = jnp.where(row < _HALF, 0.0, math.pi / 2).astype(jnp.float32)
    args = freq_col * t_row - shift_col
    n = jnp.round(args * jnp.float32(1.0 / (2.0 * math.pi)))
    r = args - n * jnp.float32(6.28125)
    r = r - n * jnp.float32(1.9353071795864769e-03)
    s = r * r
    tT = jnp.float32(-2.2398469402767916e-07)
    tT = tT * s + jnp.float32(2.430807671139143e-05)
    tT = tT * s + jnp.float32(-1.3867885560937686e-03)
    tT = tT * s + jnp.float32(4.1662991555676473e-02)
    tT = tT * s + jnp.float32(-4.999981914909368e-01)
    tT = tT * s + jnp.float32(1.0)

    temb = jax.lax.dot_general(tT, Wstep_ref[:],
                               (((0,), (0,)), ((), ())),
                               preferred_element_type=jnp.float32)
    temb = temb + bstep_ref[:]

    h = jnp.dot(x_ref[:], W1_ref[0:D, :], preferred_element_type=jnp.float32)
    h += jnp.dot(c_ref[:], W1_ref[D:2 * D, :], preferred_element_type=jnp.float32)
    h += jnp.dot(temb, W1_ref[2 * D:3 * D, :], preferred_element_type=jnp.float32)
    h += b1_ref[:]
    h = jnp.where(h > 0, h, 0.01 * h)

    out = jnp.dot(h, W2_ref[:], preferred_element_type=jnp.float32)
    out_ref[:] = out + b2_ref[:]


@jax.jit
def kernel(x, t, c, W_step, b_step, W1, b1, W2, b2):
    bx = 2048
    grid = (B // bx,)

    tf = t.astype(jnp.float32).reshape(1, B)
    out = pl.pallas_call(
        _fused_kernel,
        grid=grid,
        in_specs=[
            pl.BlockSpec((bx, D), lambda i: (i, 0)),
            pl.BlockSpec((bx, D), lambda i: (i, 0)),
            pl.BlockSpec((1, bx), lambda i: (0, i)),
            pl.BlockSpec((D, D), lambda i: (0, 0)),
            pl.BlockSpec((1, D), lambda i: (0, 0)),
            pl.BlockSpec((3 * D, 3 * D), lambda i: (0, 0)),
            pl.BlockSpec((1, 3 * D), lambda i: (0, 0)),
            pl.BlockSpec((3 * D, D), lambda i: (0, 0)),
            pl.BlockSpec((1, D), lambda i: (0, 0)),
        ],
        out_specs=pl.BlockSpec((bx, D), lambda i: (i, 0)),
        out_shape=jax.ShapeDtypeStruct((B, D), jnp.float32),
    )(x, c, tf, W_step, b_step.reshape(1, D),
      W1, b1.reshape(1, 3 * D), W2, b2.reshape(1, D))
    return out
